# baseline (speedup 1.0000x reference)
import jax
import jax.numpy as jnp
from jax import lax
from jax.experimental import pallas as pl
from jax.experimental.pallas import tpu as pltpu
from jax.experimental.pallas import tpu_sc as plsc

SIZE = 1000000
DIM = 64
BATCH = 16384
L = 16
NW = 32
RPW = SIZE // NW
NVR = BATCH // L
CW = 128
NCH = BATCH // CW

_mesh = plsc.VectorSubcoreMesh(core_axis_name="c", subcore_axis_name="s")


def _sc_scatter_body(idx_hbm, data_hbm, out_hbm, idxbuf, stamp, idxf, jf,
                     rows, gsem, ssem):
    wid = lax.axis_index("s") * 2 + lax.axis_index("c")
    base = wid * RPW
    pltpu.sync_copy(idx_hbm, idxbuf)
    iota = lax.iota(jnp.int32, L)
    zero = jnp.zeros((L,), jnp.int32)

    def p1(v, carry):
        ids = idxbuf[pl.ds(v * L, L)]
        jv = v * L + iota
        m = (ids >= base) & (ids < base + RPW)
        rloc = jnp.clip(ids - base, 0, RPW - 1)
        plsc.store_scatter(stamp, [rloc], jv, mask=m)
        return carry

    lax.fori_loop(0, NVR, p1, jnp.int32(0))

    def p2(v, carry):
        n, lastid, lastj = carry
        ids = idxbuf[pl.ds(v * L, L)]
        jv = v * L + iota
        m = (ids >= base) & (ids < base + RPW)
        rloc = jnp.clip(ids - base, 0, RPW - 1)
        w = plsc.load_gather(stamp, [rloc], mask=m)
        keep = m & (w == jv)
        ki = keep.astype(jnp.int32)
        cs = plsc.cumsum(ki)
        cnt = jnp.sum(ki)
        pos = n + cs - 1
        posc = jnp.clip(pos, 0, BATCH - 1)
        row = jnp.right_shift(posc, 7)
        col = posc & (CW - 1)
        plsc.store_scatter(idxf, [row, col], ids, mask=keep)
        plsc.store_scatter(jf, [row, col], jv, mask=keep)
        nn = n + cnt
        sel = keep & (pos == nn - 1)
        lid = jnp.sum(jnp.where(sel, ids, zero))
        lj = jnp.sum(jnp.where(sel, jv, zero))
        has = cnt > 0
        return (nn, jnp.where(has, lid, lastid), jnp.where(has, lj, lastj))

    n, lastid, lastj = lax.fori_loop(
        0, NVR, p2, (jnp.int32(0), jnp.int32(0), jnp.int32(0)))

    ntot = jnp.bitwise_and(n + (CW - 1), -CW)
    lid_v = zero + lastid
    lj_v = zero + lastj
    for p in range(CW // L):
        lanepos = n + p * L + iota
        mp = lanepos < ntot
        pc = jnp.clip(lanepos, 0, BATCH - 1)
        row = jnp.right_shift(pc, 7)
        col = pc & (CW - 1)
        plsc.store_scatter(idxf, [row, col], lid_v, mask=mp)
        plsc.store_scatter(jf, [row, col], lj_v, mask=mp)

    nch = jnp.right_shift(ntot, 7)

    def dma(ch, carry):
        pltpu.async_copy(data_hbm.at[jf.at[ch]], rows, gsem).wait()
        pltpu.async_copy(rows, out_hbm.at[idxf.at[ch]], ssem).wait()
        return carry

    lax.fori_loop(0, nch, dma, jnp.int32(0))


_sc_scatter = pl.kernel(
    _sc_scatter_body,
    out_type=(),
    mesh=_mesh,
    compiler_params=pltpu.CompilerParams(
        needs_layout_passes=False, use_tc_tiling_on_sc=False),
    scratch_types=[
        pltpu.VMEM((BATCH,), jnp.int32),
        pltpu.VMEM((RPW,), jnp.int32),
        pltpu.VMEM((NCH, CW), jnp.int32),
        pltpu.VMEM((NCH, CW), jnp.int32),
        pltpu.VMEM((CW, DIM), jnp.float32),
        pltpu.SemaphoreType.DMA,
        pltpu.SemaphoreType.DMA,
    ],
)


def kernel(bank, indices, data_memory):
    out_ref = jax.new_ref(bank)
    _sc_scatter(indices.astype(jnp.int32), data_memory, out_ref)
    return jax.freeze(out_ref)
